# Initial kernel scaffold; baseline (speedup 1.0000x reference)
import jax
import jax.numpy as jnp
from jax.experimental import pallas as pl


def kernel(pos_h, pos_t, pos_r, neg_h, neg_t, neg_r, ent_embeddings, rel_embeddings):
    raise NotImplementedError("write your pallas kernel here")


# speedup vs baseline: 1.2332x; 1.2332x over previous
import functools

import jax
import jax.numpy as jnp
from jax import lax
from jax.experimental import pallas as pl
from jax.experimental.pallas import tpu as pltpu
from jax.experimental.pallas import tpu_sc as plsc

ENT = 1000000
REL = 1000
D = 64
MARGIN = 1.0
B = 4096
NEG = 25

NC = 2
NS = 16
L = 16
NW = NC * NS

PAIRS = NEG + 1
B_PER_W = B // NW
E_PER_C = 4
CHUNK = E_PER_C * PAIRS
NCHUNK = B_PER_W // E_PER_C
KD = D // L


def _body(h_hbm, t_hbm, r_hbm, ent_hbm, rel_hbm, out_hbm,
          h_idx, t_idx, r_idx, bh, bt, br, loss_v,
          semh, semt, semr):
    wid = lax.axis_index("s") * NC + lax.axis_index("c")
    iota = lax.iota(jnp.int32, L)

    pltpu.sync_copy(h_hbm.at[wid], h_idx)
    pltpu.sync_copy(t_hbm.at[wid], t_idx)
    pltpu.sync_copy(r_hbm.at[wid], r_idx)

    def pair_acc(acc, p):
        for k in range(KD):
            sl = pl.ds(k * L, L)
            acc = acc + jnp.abs(bh[p, sl] - bt[p, sl] + br[p, sl])
        return acc

    def chunk_body(c, loss):
        ch = pltpu.async_copy(ent_hbm.at[h_idx.at[c]], bh, semh)
        ct = pltpu.async_copy(ent_hbm.at[t_idx.at[c]], bt, semt)
        cr = pltpu.async_copy(rel_hbm.at[r_idx.at[c]], br, semr)
        ch.wait()
        ct.wait()
        cr.wait()

        def e_body(e, loss):
            base = e * PAIRS
            snv = jnp.zeros((L,), jnp.float32)
            for j in range(NEG):
                snv = pair_acc(snv, base + j)
            spv = pair_acc(jnp.zeros((L,), jnp.float32), base + NEG)
            sn = lax.reduce_sum_p.bind(snv, axes=(0,))
            sp = lax.reduce_sum_p.bind(spv, axes=(0,))
            return loss + jnp.maximum(sp - sn * (1.0 / NEG) + MARGIN, 0.0)

        return lax.fori_loop(0, E_PER_C, e_body, loss)

    loss = lax.fori_loop(0, NCHUNK, chunk_body, jnp.float32(0.0))

    loss_v[...] = jnp.where(iota == 0, loss, 0.0)
    pltpu.sync_copy(loss_v, out_hbm.at[wid])


def kernel(pos_h, pos_t, pos_r, neg_h, neg_t, neg_r, ent_embeddings, rel_embeddings):
    h3 = jnp.concatenate([neg_h, pos_h], axis=1).reshape(NW, NCHUNK, CHUNK)
    t3 = jnp.concatenate([neg_t, pos_t], axis=1).reshape(NW, NCHUNK, CHUNK)
    r3 = jnp.concatenate([neg_r, pos_r], axis=1).reshape(NW, NCHUNK, CHUNK)

    run = functools.partial(
        pl.kernel,
        mesh=plsc.VectorSubcoreMesh(core_axis_name="c", subcore_axis_name="s"),
        compiler_params=pltpu.CompilerParams(
            needs_layout_passes=False, use_tc_tiling_on_sc=False),
        out_type=jax.ShapeDtypeStruct((NW, L), jnp.float32),
        scratch_types=[
            pltpu.VMEM((NCHUNK, CHUNK), jnp.int32),
            pltpu.VMEM((NCHUNK, CHUNK), jnp.int32),
            pltpu.VMEM((NCHUNK, CHUNK), jnp.int32),
            pltpu.VMEM((CHUNK, D), jnp.float32),
            pltpu.VMEM((CHUNK, D), jnp.float32),
            pltpu.VMEM((CHUNK, D), jnp.float32),
            pltpu.VMEM((L,), jnp.float32),
            pltpu.SemaphoreType.DMA,
            pltpu.SemaphoreType.DMA,
            pltpu.SemaphoreType.DMA,
        ],
    )(_body)

    partials = run(h3, t3, r3, ent_embeddings, rel_embeddings)
    return jnp.sum(partials)


# speedup vs baseline: 1.6267x; 1.3191x over previous
import functools

import jax
import jax.numpy as jnp
from jax import lax
from jax.experimental import pallas as pl
from jax.experimental.pallas import tpu as pltpu
from jax.experimental.pallas import tpu_sc as plsc

ENT = 1000000
REL = 1000
D = 64
MARGIN = 1.0
B = 4096
NEG = 25

NC = 2
NS = 16
L = 16
NW = NC * NS

PAIRS = NEG + 1
B_PER_W = B // NW
E_PER_C = 4
CHUNK = E_PER_C * PAIRS
NCHUNK = B_PER_W // E_PER_C
KD = D // L


def _body(h_hbm, t_hbm, r_hbm, ent_hbm, rel_hbm, out_hbm,
          h_idx, t_idx, r_idx, bh, bt, br, loss_v,
          semh, semt, semr):
    wid = lax.axis_index("s") * NC + lax.axis_index("c")
    iota = lax.iota(jnp.int32, L)

    pltpu.sync_copy(h_hbm.at[wid], h_idx)
    pltpu.sync_copy(t_hbm.at[wid], t_idx)
    pltpu.sync_copy(r_hbm.at[wid], r_idx)

    def row_copies(c, e, idx_ref, table, buf, sem):
        eoff = e * PAIRS
        v0 = idx_ref.at[c][pl.ds(eoff, L)]
        v1 = idx_ref.at[c][pl.ds(eoff + PAIRS - L, L)]
        copies = []
        for j in range(PAIRS):
            row = v0[j] if j < L else v1[j - (PAIRS - L)]
            copies.append(pltpu.async_copy(
                table.at[pl.ds(row, 1)], buf.at[pl.ds(eoff + j, 1)], sem))
        return copies

    def pair_acc(acc, p):
        for k in range(KD):
            sl = pl.ds(k * L, L)
            acc = acc + jnp.abs(bh[p, sl] - bt[p, sl] + br[p, sl])
        return acc

    def chunk_body(c, loss):
        def issue(e, carry):
            row_copies(c, e, h_idx, ent_hbm, bh, semh)
            row_copies(c, e, t_idx, ent_hbm, bt, semt)
            row_copies(c, e, r_idx, rel_hbm, br, semr)
            return carry

        lax.fori_loop(0, E_PER_C, issue, jnp.int32(0))
        for sem, table, buf in ((semh, ent_hbm, bh), (semt, ent_hbm, bt),
                                (semr, rel_hbm, br)):
            for _ in range(CHUNK):
                pltpu.make_async_copy(table.at[pl.ds(0, 1)],
                                      buf.at[pl.ds(0, 1)], sem).wait()

        def e_body(e, loss):
            base = e * PAIRS
            snv = jnp.zeros((L,), jnp.float32)
            for j in range(NEG):
                snv = pair_acc(snv, base + j)
            spv = pair_acc(jnp.zeros((L,), jnp.float32), base + NEG)
            sn = lax.reduce_sum_p.bind(snv, axes=(0,))
            sp = lax.reduce_sum_p.bind(spv, axes=(0,))
            return loss + jnp.maximum(sp - sn * (1.0 / NEG) + MARGIN, 0.0)

        return lax.fori_loop(0, E_PER_C, e_body, loss)

    loss = lax.fori_loop(0, NCHUNK, chunk_body, jnp.float32(0.0))

    loss_v[...] = jnp.where(iota == 0, loss, 0.0)
    pltpu.sync_copy(loss_v, out_hbm.at[wid])


def kernel(pos_h, pos_t, pos_r, neg_h, neg_t, neg_r, ent_embeddings, rel_embeddings):
    h3 = jnp.concatenate([neg_h, pos_h], axis=1).reshape(NW, NCHUNK, CHUNK)
    t3 = jnp.concatenate([neg_t, pos_t], axis=1).reshape(NW, NCHUNK, CHUNK)
    r3 = jnp.concatenate([neg_r, pos_r], axis=1).reshape(NW, NCHUNK, CHUNK)

    run = functools.partial(
        pl.kernel,
        mesh=plsc.VectorSubcoreMesh(core_axis_name="c", subcore_axis_name="s"),
        compiler_params=pltpu.CompilerParams(
            needs_layout_passes=False, use_tc_tiling_on_sc=True),
        out_type=jax.ShapeDtypeStruct((NW, L), jnp.float32),
        scratch_types=[
            pltpu.VMEM((NCHUNK, CHUNK), jnp.int32),
            pltpu.VMEM((NCHUNK, CHUNK), jnp.int32),
            pltpu.VMEM((NCHUNK, CHUNK), jnp.int32),
            pltpu.VMEM((CHUNK, D), jnp.float32),
            pltpu.VMEM((CHUNK, D), jnp.float32),
            pltpu.VMEM((CHUNK, D), jnp.float32),
            pltpu.VMEM((L,), jnp.float32),
            pltpu.SemaphoreType.DMA,
            pltpu.SemaphoreType.DMA,
            pltpu.SemaphoreType.DMA,
        ],
    )(_body)

    partials = run(h3, t3, r3, ent_embeddings, rel_embeddings)
    return jnp.sum(partials)
